# baseline (speedup 1.0000x reference)
import functools

import jax
import jax.numpy as jnp
from jax import lax
from jax.experimental import pallas as pl
from jax.experimental.pallas import tpu as pltpu
from jax.experimental.pallas import tpu_sc as plsc

N = 10000
E = 320000
NC = 2
NS = 16
NW = NC * NS
CHUNK = 125
NB = 8
PF = 4
CH = E // (NW * CHUNK)
ACC_ROWS = 10240
ZCH = 64
ZPT = ACC_ROWS // NS
WPT = 624
WTAIL = N - NS * WPT


def _make_agg(D):
    mesh = plsc.VectorSubcoreMesh(core_axis_name="c", subcore_axis_name="s")

    @functools.partial(
        pl.kernel,
        out_type=jax.ShapeDtypeStruct((NC, N, D), jnp.bfloat16),
        mesh=mesh,
        compiler_params=pltpu.CompilerParams(use_tc_tiling_on_sc=False),
        scratch_types=(
            [
                pltpu.VMEM((CH, CHUNK), jnp.int32),
                pltpu.VMEM((CH, CHUNK), jnp.int32),
                pltpu.VMEM((ZCH, D), jnp.bfloat16),
                pltpu.VMEM_SHARED((ACC_ROWS, D), jnp.bfloat16),
            ]
            + [pltpu.VMEM((CHUNK, D), jnp.bfloat16) for _ in range(NB)]
            + [pltpu.SemaphoreType.DMA for _ in range(2 * NB)]
        ),
    )
    def agg(table_hbm, srcs_hbm, dsts_hbm, zeros_hbm, out_hbm, si, di, zb, acc, *rs):
        rows = rs[:NB]
        gsem = rs[NB:2 * NB]
        ssem = rs[2 * NB:]
        c = lax.axis_index("c")
        s = lax.axis_index("s")
        wid = c * NS + s

        pltpu.sync_copy(zeros_hbm, zb)
        zbase = s * ZPT
        for k in range(ZPT // ZCH):
            pltpu.sync_copy(zb, acc.at[pl.ds(zbase + k * ZCH, ZCH)])
        rem = ZPT % ZCH
        if rem:
            pltpu.sync_copy(zb.at[pl.ds(0, rem)],
                            acc.at[pl.ds(zbase + (ZPT // ZCH) * ZCH, rem)])
        plsc.subcore_barrier()

        pltpu.sync_copy(srcs_hbm.at[wid], si)
        pltpu.sync_copy(dsts_hbm.at[wid], di)

        for b in range(PF):
            pltpu.async_copy(table_hbm.at[si.at[b]], rows[b], gsem[b])

        @pl.loop(0, CH, step=NB)
        def _(j0):
            for k in range(NB):
                j = j0 + k
                sg = (k + PF) % NB

                @pl.when(j + PF < CH)
                def _():
                    @pl.when(j + PF >= NB)
                    def _():
                        pltpu.make_async_copy(
                            rows[sg], acc.at[di.at[j + PF - NB]], ssem[sg]).wait()
                    pltpu.async_copy(table_hbm.at[si.at[j + PF]], rows[sg],
                                     gsem[sg])

                pltpu.make_async_copy(table_hbm.at[si.at[j]], rows[k],
                                      gsem[k]).wait()
                pltpu.async_copy(rows[k], acc.at[di.at[j]], ssem[k], add=True)

        for b in range(NB):
            pltpu.make_async_copy(rows[b], acc.at[di.at[CH - NB + b]],
                                  ssem[b]).wait()

        plsc.subcore_barrier()

        wbase = s * WPT
        pltpu.sync_copy(acc.at[pl.ds(wbase, WPT)],
                        out_hbm.at[c].at[pl.ds(wbase, WPT)])

        @pl.when(s == NS - 1)
        def _():
            pltpu.sync_copy(acc.at[pl.ds(NS * WPT, WTAIL)],
                            out_hbm.at[c].at[pl.ds(NS * WPT, WTAIL)])

    return agg


_agg128 = _make_agg(128)
_agg64 = _make_agg(64)

_BR = 1000


def _fused_body(p_ref, w1_ref, c1_ref, w2_ref, o_ref):
    h = p_ref[0].astype(jnp.float32) + p_ref[1].astype(jnp.float32)
    h = jnp.dot(h, w1_ref[...], preferred_element_type=jnp.float32)
    h = jnp.maximum(h + c1_ref[...], 0.0)
    y = jnp.dot(h, w2_ref[...], preferred_element_type=jnp.float32)
    o_ref[...] = y.astype(jnp.bfloat16)


_fused = pl.pallas_call(
    _fused_body,
    grid=(N // _BR,),
    in_specs=[
        pl.BlockSpec((NC, _BR, 128), lambda i: (0, i, 0)),
        pl.BlockSpec((128, 128), lambda i: (0, 0)),
        pl.BlockSpec((1, 128), lambda i: (0, 0)),
        pl.BlockSpec((128, 64), lambda i: (0, 0)),
    ],
    out_specs=pl.BlockSpec((_BR, 64), lambda i: (i, 0)),
    out_shape=jax.ShapeDtypeStruct((N, 64), jnp.bfloat16),
)


def _final_body(q_ref, b2_ref, o_ref):
    v = q_ref[0].astype(jnp.float32) + q_ref[1].astype(jnp.float32) + b2_ref[...]
    nrm = jnp.sqrt(jnp.sum(v * v, axis=1, keepdims=True))
    o_ref[...] = v / jnp.maximum(nrm, 1e-12)


_final = pl.pallas_call(
    _final_body,
    grid=(N // _BR,),
    in_specs=[
        pl.BlockSpec((NC, _BR, 64), lambda i: (0, i, 0)),
        pl.BlockSpec((1, 64), lambda i: (0, 0)),
    ],
    out_specs=pl.BlockSpec((_BR, 64), lambda i: (i, 0)),
    out_shape=jax.ShapeDtypeStruct((N, 64), jnp.float32),
)


def kernel(x, edge_index, W1, b1, W2, b2, bn_gamma, bn_beta, bn_mean, bn_var):
    scale = bn_gamma * lax.rsqrt(bn_var + 1e-5)
    W1e = W1 * scale[None, :]
    c1 = ((b1 - bn_mean) * scale + bn_beta)[None, :]

    ei = edge_index.reshape(2, NW, CH, CHUNK)
    srcs = ei[0]
    dsts = ei[1]

    z128 = jnp.zeros((ZCH, 128), jnp.bfloat16)
    z64 = jnp.zeros((ZCH, 64), jnp.bfloat16)

    p = _agg128(x.astype(jnp.bfloat16), srcs, dsts, z128)
    h2 = _fused(p, W1e, c1, W2)
    q = _agg64(h2, srcs, dsts, z64)
    return _final(q, b2[None, :])


# speedup vs baseline: 1.0190x; 1.0190x over previous
import functools

import jax
import jax.numpy as jnp
from jax import lax
from jax.experimental import pallas as pl
from jax.experimental.pallas import tpu as pltpu
from jax.experimental.pallas import tpu_sc as plsc

N = 10000
E = 320000
NC = 2
NS = 16
NW = NC * NS
CHUNK = 128
NB = 8
PF = 4
CH = (-(-E // (NW * CHUNK)) + NB - 1) // NB * NB
EPT = CH * CHUNK
PAD = NW * EPT - E
ACC_ROWS = 10240
ZCH = 64
ZPT = ACC_ROWS // NS
WPT = 624
WTAIL = N - NS * WPT


def _make_agg(D):
    mesh = plsc.VectorSubcoreMesh(core_axis_name="c", subcore_axis_name="s")

    @functools.partial(
        pl.kernel,
        out_type=jax.ShapeDtypeStruct((NC, N, D), jnp.bfloat16),
        mesh=mesh,
        compiler_params=pltpu.CompilerParams(use_tc_tiling_on_sc=False),
        scratch_types=(
            [
                pltpu.VMEM((CH, CHUNK), jnp.int32),
                pltpu.VMEM((CH, CHUNK), jnp.int32),
                pltpu.VMEM((ZCH, D), jnp.bfloat16),
                pltpu.VMEM_SHARED((ACC_ROWS, D), jnp.bfloat16),
            ]
            + [pltpu.VMEM((CHUNK, D), jnp.bfloat16) for _ in range(NB)]
            + [pltpu.SemaphoreType.DMA for _ in range(2 * NB)]
        ),
    )
    def agg(table_hbm, srcs_hbm, dsts_hbm, zeros_hbm, out_hbm, si, di, zb, acc, *rs):
        rows = rs[:NB]
        gsem = rs[NB:2 * NB]
        ssem = rs[2 * NB:]
        c = lax.axis_index("c")
        s = lax.axis_index("s")
        wid = c * NS + s

        pltpu.sync_copy(zeros_hbm, zb)
        zbase = s * ZPT
        for k in range(ZPT // ZCH):
            pltpu.sync_copy(zb, acc.at[pl.ds(zbase + k * ZCH, ZCH)])
        rem = ZPT % ZCH
        if rem:
            pltpu.sync_copy(zb.at[pl.ds(0, rem)],
                            acc.at[pl.ds(zbase + (ZPT // ZCH) * ZCH, rem)])
        plsc.subcore_barrier()

        pltpu.sync_copy(srcs_hbm.at[wid], si)
        pltpu.sync_copy(dsts_hbm.at[wid], di)

        for b in range(PF):
            pltpu.async_copy(table_hbm.at[si.at[b]], rows[b], gsem[b])

        @pl.loop(0, CH, step=NB)
        def _(j0):
            for k in range(NB):
                j = j0 + k
                sg = (k + PF) % NB

                @pl.when(j + PF < CH)
                def _():
                    @pl.when(j + PF >= NB)
                    def _():
                        pltpu.make_async_copy(
                            rows[sg], acc.at[di.at[j + PF - NB]], ssem[sg]).wait()
                    pltpu.async_copy(table_hbm.at[si.at[j + PF]], rows[sg],
                                     gsem[sg])

                pltpu.make_async_copy(table_hbm.at[si.at[j]], rows[k],
                                      gsem[k]).wait()
                pltpu.async_copy(rows[k], acc.at[di.at[j]], ssem[k], add=True)

        for b in range(NB):
            pltpu.make_async_copy(rows[b], acc.at[di.at[CH - NB + b]],
                                  ssem[b]).wait()

        plsc.subcore_barrier()

        wbase = s * WPT
        pltpu.sync_copy(acc.at[pl.ds(wbase, WPT)],
                        out_hbm.at[c].at[pl.ds(wbase, WPT)])

        @pl.when(s == NS - 1)
        def _():
            pltpu.sync_copy(acc.at[pl.ds(NS * WPT, WTAIL)],
                            out_hbm.at[c].at[pl.ds(NS * WPT, WTAIL)])

    return agg


_agg128 = _make_agg(128)
_agg64 = _make_agg(64)

_BR = 1000


def _fused_body(p_ref, w1_ref, c1_ref, w2_ref, o_ref):
    h = jnp.dot(p_ref[...], w1_ref[...], preferred_element_type=jnp.float32)
    h = jnp.maximum(h + c1_ref[...], 0.0)
    y = jnp.dot(h, w2_ref[...], preferred_element_type=jnp.float32)
    o_ref[...] = y.astype(jnp.bfloat16)


_fused = pl.pallas_call(
    _fused_body,
    grid=(N // _BR,),
    in_specs=[
        pl.BlockSpec((_BR, 128), lambda i: (i, 0)),
        pl.BlockSpec((128, 128), lambda i: (0, 0)),
        pl.BlockSpec((1, 128), lambda i: (0, 0)),
        pl.BlockSpec((128, 64), lambda i: (0, 0)),
    ],
    out_specs=pl.BlockSpec((_BR, 64), lambda i: (i, 0)),
    out_shape=jax.ShapeDtypeStruct((N, 64), jnp.bfloat16),
)


def _final_body(q_ref, b2_ref, o_ref):
    v = q_ref[...] + b2_ref[...]
    nrm = jnp.sqrt(jnp.sum(v * v, axis=1, keepdims=True))
    o_ref[...] = v / jnp.maximum(nrm, 1e-12)


_final = pl.pallas_call(
    _final_body,
    grid=(N // _BR,),
    in_specs=[
        pl.BlockSpec((_BR, 64), lambda i: (i, 0)),
        pl.BlockSpec((1, 64), lambda i: (0, 0)),
    ],
    out_specs=pl.BlockSpec((_BR, 64), lambda i: (i, 0)),
    out_shape=jax.ShapeDtypeStruct((N, 64), jnp.float32),
)


def kernel(x, edge_index, W1, b1, W2, b2, bn_gamma, bn_beta, bn_mean, bn_var):
    scale = bn_gamma * lax.rsqrt(bn_var + 1e-5)
    W1e = W1 * scale[None, :]
    c1 = ((b1 - bn_mean) * scale + bn_beta)[None, :]

    pad_iota = jnp.arange(PAD, dtype=jnp.int32)
    src = jnp.concatenate([edge_index[0], pad_iota % N])
    dst = jnp.concatenate([edge_index[1], N + pad_iota % (ACC_ROWS - N)])
    srcs = src.reshape(NW, CH, CHUNK)
    dsts = dst.reshape(NW, CH, CHUNK)

    z128 = jnp.zeros((ZCH, 128), jnp.bfloat16)
    z64 = jnp.zeros((ZCH, 64), jnp.bfloat16)

    p = _agg128(x.astype(jnp.bfloat16), srcs, dsts, z128)
    psum = p[0].astype(jnp.float32) + p[1].astype(jnp.float32)
    h2 = _fused(psum, W1e, c1, W2)
    q = _agg64(h2, srcs, dsts, z64)
    qsum = q[0].astype(jnp.float32) + q[1].astype(jnp.float32)
    return _final(qsum, b2[None, :])


# speedup vs baseline: 1.0679x; 1.0480x over previous
import functools

import jax
import jax.numpy as jnp
from jax import lax
from jax.experimental import pallas as pl
from jax.experimental.pallas import tpu as pltpu
from jax.experimental.pallas import tpu_sc as plsc

N = 10000
E = 320000
NC = 2
NS = 16
NW = NC * NS
CHUNK = 128
NB = 8
PF = 6
CH = (-(-E // (NW * CHUNK)) + NB - 1) // NB * NB
EPT = CH * CHUNK
PAD = NW * EPT - E
ACC_ROWS = 10240
ZCH = 64
ZPT = ACC_ROWS // NS
WPT = 624
WTAIL = N - NS * WPT


def _make_agg(D):
    mesh = plsc.VectorSubcoreMesh(core_axis_name="c", subcore_axis_name="s")

    @functools.partial(
        pl.kernel,
        out_type=jax.ShapeDtypeStruct((NC, N, D), jnp.bfloat16),
        mesh=mesh,
        compiler_params=pltpu.CompilerParams(use_tc_tiling_on_sc=False),
        scratch_types=(
            [
                pltpu.VMEM((CH, CHUNK), jnp.int32),
                pltpu.VMEM((CH, CHUNK), jnp.int32),
                pltpu.VMEM((ZCH, D), jnp.bfloat16),
                pltpu.VMEM_SHARED((ACC_ROWS, D), jnp.bfloat16),
            ]
            + [pltpu.VMEM((CHUNK, D), jnp.bfloat16) for _ in range(NB)]
            + [pltpu.SemaphoreType.DMA for _ in range(2 * NB)]
        ),
    )
    def agg(table_hbm, srcs_hbm, dsts_hbm, zeros_hbm, out_hbm, si, di, zb, acc, *rs):
        rows = rs[:NB]
        gsem = rs[NB:2 * NB]
        ssem = rs[2 * NB:]
        c = lax.axis_index("c")
        s = lax.axis_index("s")
        wid = c * NS + s

        pltpu.sync_copy(zeros_hbm, zb)
        zbase = s * ZPT
        for k in range(ZPT // ZCH):
            pltpu.sync_copy(zb, acc.at[pl.ds(zbase + k * ZCH, ZCH)])
        rem = ZPT % ZCH
        if rem:
            pltpu.sync_copy(zb.at[pl.ds(0, rem)],
                            acc.at[pl.ds(zbase + (ZPT // ZCH) * ZCH, rem)])
        plsc.subcore_barrier()

        pltpu.sync_copy(srcs_hbm.at[wid], si)
        pltpu.sync_copy(dsts_hbm.at[wid], di)

        for b in range(PF):
            pltpu.async_copy(table_hbm.at[si.at[b]], rows[b], gsem[b])

        @pl.loop(0, CH, step=NB)
        def _(j0):
            for k in range(NB):
                j = j0 + k
                sg = (k + PF) % NB

                @pl.when(j + PF < CH)
                def _():
                    @pl.when(j + PF >= NB)
                    def _():
                        pltpu.make_async_copy(
                            rows[sg], acc.at[di.at[j + PF - NB]], ssem[sg]).wait()
                    pltpu.async_copy(table_hbm.at[si.at[j + PF]], rows[sg],
                                     gsem[sg])

                pltpu.make_async_copy(table_hbm.at[si.at[j]], rows[k],
                                      gsem[k]).wait()
                pltpu.async_copy(rows[k], acc.at[di.at[j]], ssem[k], add=True)

        for b in range(NB):
            pltpu.make_async_copy(rows[b], acc.at[di.at[CH - NB + b]],
                                  ssem[b]).wait()

        plsc.subcore_barrier()

        wbase = s * WPT
        pltpu.sync_copy(acc.at[pl.ds(wbase, WPT)],
                        out_hbm.at[c].at[pl.ds(wbase, WPT)])

        @pl.when(s == NS - 1)
        def _():
            pltpu.sync_copy(acc.at[pl.ds(NS * WPT, WTAIL)],
                            out_hbm.at[c].at[pl.ds(NS * WPT, WTAIL)])

    return agg


_agg128 = _make_agg(128)
_agg64 = _make_agg(64)

_BR = 1000


def _fused_body(p_ref, w1_ref, c1_ref, w2_ref, o_ref):
    h = p_ref[0].astype(jnp.float32) + p_ref[1].astype(jnp.float32)
    h = jnp.dot(h, w1_ref[...], preferred_element_type=jnp.float32)
    h = jnp.maximum(h + c1_ref[...], 0.0)
    y = jnp.dot(h, w2_ref[...], preferred_element_type=jnp.float32)
    o_ref[...] = y.astype(jnp.bfloat16)


_fused = pl.pallas_call(
    _fused_body,
    grid=(N // _BR,),
    in_specs=[
        pl.BlockSpec((NC, _BR, 128), lambda i: (0, i, 0)),
        pl.BlockSpec((128, 128), lambda i: (0, 0)),
        pl.BlockSpec((1, 128), lambda i: (0, 0)),
        pl.BlockSpec((128, 64), lambda i: (0, 0)),
    ],
    out_specs=pl.BlockSpec((_BR, 64), lambda i: (i, 0)),
    out_shape=jax.ShapeDtypeStruct((N, 64), jnp.bfloat16),
)


def _final_body(q_ref, b2_ref, o_ref):
    v = q_ref[0].astype(jnp.float32) + q_ref[1].astype(jnp.float32) + b2_ref[...]
    nrm = jnp.sqrt(jnp.sum(v * v, axis=1, keepdims=True))
    o_ref[...] = v / jnp.maximum(nrm, 1e-12)


_final = pl.pallas_call(
    _final_body,
    grid=(N // _BR,),
    in_specs=[
        pl.BlockSpec((NC, _BR, 64), lambda i: (0, i, 0)),
        pl.BlockSpec((1, 64), lambda i: (0, 0)),
    ],
    out_specs=pl.BlockSpec((_BR, 64), lambda i: (i, 0)),
    out_shape=jax.ShapeDtypeStruct((N, 64), jnp.float32),
)


def kernel(x, edge_index, W1, b1, W2, b2, bn_gamma, bn_beta, bn_mean, bn_var):
    scale = bn_gamma * lax.rsqrt(bn_var + 1e-5)
    W1e = W1 * scale[None, :]
    c1 = ((b1 - bn_mean) * scale + bn_beta)[None, :]

    pad_iota = jnp.arange(PAD, dtype=jnp.int32)
    src = jnp.concatenate([edge_index[0], pad_iota % N])
    dst = jnp.concatenate([edge_index[1], N + pad_iota % (ACC_ROWS - N)])
    srcs = src.reshape(NW, CH, CHUNK)
    dsts = dst.reshape(NW, CH, CHUNK)

    z128 = jnp.zeros((ZCH, 128), jnp.bfloat16)
    z64 = jnp.zeros((ZCH, 64), jnp.bfloat16)

    p = _agg128(x.astype(jnp.bfloat16), srcs, dsts, z128)
    h2 = _fused(p, W1e, c1, W2)
    q = _agg64(h2, srcs, dsts, z64)
    return _final(q, b2[None, :])
